# baseline (speedup 1.0000x reference)
import functools

import jax
import jax.numpy as jnp
from jax import lax
from jax.experimental import pallas as pl
from jax.experimental.pallas import tpu as pltpu
from jax.experimental.pallas import tpu_sc as plsc

NUM_CLASSES = 40
OUT_CHANNELS = 512
N = 100000

LANES = 16
NUM_WORKERS = 32
CHUNK = 80
NUM_CHUNKS = N // CHUNK
CHUNKS_PER_WORKER = -(-NUM_CHUNKS // NUM_WORKERS)


def _sc_body(y_hbm, mask_hbm, table_hbm, out_hbm, y_v, m_v, idx_v, rows_v, sem):
    nc = plsc.get_sparse_core_info().num_cores
    wid = lax.axis_index("s") * nc + lax.axis_index("c")

    def chunk_step(t, carry):
        k = wid + t * NUM_WORKERS

        @pl.when(k < NUM_CHUNKS)
        def _():
            base = k * CHUNK
            pltpu.sync_copy(y_hbm.at[pl.ds(base, CHUNK)], y_v)
            pltpu.sync_copy(mask_hbm.at[pl.ds(base, CHUNK)], m_v)
            for j in range(CHUNK // LANES):
                sl = pl.ds(j * LANES, LANES)
                yv = y_v[sl]
                mv = m_v[sl]
                idx_v[sl] = jnp.where(mv != 0, yv, NUM_CLASSES)
            pltpu.async_copy(table_hbm.at[idx_v], rows_v, sem).wait()
            pltpu.sync_copy(rows_v, out_hbm.at[pl.ds(base, CHUNK)])

        return carry

    lax.fori_loop(0, CHUNKS_PER_WORKER, chunk_step, 0)


@jax.jit
def _masked_lookup(y, mask_i32, table):
    mesh = plsc.VectorSubcoreMesh(core_axis_name="c", subcore_axis_name="s")
    return pl.kernel(
        _sc_body,
        out_type=jax.ShapeDtypeStruct((N, OUT_CHANNELS), jnp.float32),
        mesh=mesh,
        scratch_types=[
            pltpu.VMEM((CHUNK,), jnp.int32),
            pltpu.VMEM((CHUNK,), jnp.int32),
            pltpu.VMEM((CHUNK,), jnp.int32),
            pltpu.VMEM((CHUNK, OUT_CHANNELS), jnp.float32),
            pltpu.SemaphoreType.DMA,
        ],
    )(y, mask_i32, table)


def kernel(y, mask, emb):
    table = jnp.concatenate(
        [emb, jnp.zeros((1, OUT_CHANNELS), dtype=emb.dtype)], axis=0
    )
    return _masked_lookup(y.astype(jnp.int32), mask.astype(jnp.int32), table)
